# baseline (speedup 1.0000x reference)
import functools

import jax
import jax.numpy as jnp
from jax import lax
from jax.experimental import pallas as pl
from jax.experimental.pallas import tpu as pltpu
from jax.experimental.pallas import tpu_sc as plsc

_ROWS = 16384
_D = 2048
_NC = 2
_NS = 16
_NW = _NC * _NS
_RPW = _ROWS // _NW
_CH = 8
_NBUF = 6
_NCH = _RPW // _CH


def _sc_copy(x_hbm, o_hbm, spmem, *sems):
    lsem = sems[:_NBUF]
    ssem = sems[_NBUF:]
    sid = lax.axis_index("s")
    wid = sid * _NC + lax.axis_index("c")
    base = wid * _RPW

    def start_load(i, slot):
        c = pltpu.make_async_copy(
            x_hbm.at[pl.ds(base + i * _CH, _CH)], spmem.at[sid, slot], lsem[slot]
        )
        c.start()
        return c

    def start_store(i, slot):
        c = pltpu.make_async_copy(
            spmem.at[sid, slot], o_hbm.at[pl.ds(base + i * _CH, _CH)], ssem[slot]
        )
        c.start()
        return c

    loads = [None] * _NBUF
    stores = [None] * _NBUF
    for j in range(_NBUF - 1):
        loads[j] = start_load(j, j)
    for i in range(_NCH):
        slot = i % _NBUF
        nxt = i + _NBUF - 1
        if nxt < _NCH:
            nslot = nxt % _NBUF
            if stores[nslot] is not None:
                stores[nslot].wait()
            loads[nslot] = start_load(nxt, nslot)
        loads[slot].wait()
        stores[slot] = start_store(i, slot)
    for j in range(_NBUF):
        stores[j].wait()


_sc_kernel = functools.partial(
    pl.kernel,
    mesh=plsc.VectorSubcoreMesh(core_axis_name="c", subcore_axis_name="s"),
    out_type=jax.ShapeDtypeStruct((_ROWS, _D), jnp.float32),
    scratch_types=(
        [pltpu.VMEM_SHARED((_NS, _NBUF, _CH, _D), jnp.float32)]
        + [pltpu.SemaphoreType.DMA] * (2 * _NBUF)
    ),
)(_sc_copy)


def kernel(x):
    b, s, d = x.shape
    x2 = x.reshape(b * s, d)
    out = _sc_kernel(x2)
    return out.reshape(b, s, d)


# speedup vs baseline: 1.0483x; 1.0483x over previous
import functools

import jax
import jax.numpy as jnp
from jax import lax
from jax.experimental import pallas as pl
from jax.experimental.pallas import tpu as pltpu
from jax.experimental.pallas import tpu_sc as plsc

_ROWS = 16384
_D = 2048
_NC = 2
_NS = 16
_NW = _NC * _NS
_RPW = _ROWS // _NW
_CH = 16
_NBUF = 3
_NCH = _RPW // _CH


def _sc_copy(x_hbm, o_hbm, spmem, *sems):
    lsem = sems[:_NBUF]
    ssem = sems[_NBUF:]
    sid = lax.axis_index("s")
    wid = sid * _NC + lax.axis_index("c")
    base = wid * _RPW

    def start_load(i, slot):
        c = pltpu.make_async_copy(
            x_hbm.at[pl.ds(base + i * _CH, _CH)], spmem.at[sid, slot], lsem[slot]
        )
        c.start()
        return c

    def start_store(i, slot):
        c = pltpu.make_async_copy(
            spmem.at[sid, slot], o_hbm.at[pl.ds(base + i * _CH, _CH)], ssem[slot]
        )
        c.start()
        return c

    loads = [None] * _NBUF
    stores = [None] * _NBUF
    for j in range(_NBUF - 1):
        loads[j] = start_load(j, j)
    for i in range(_NCH):
        slot = i % _NBUF
        nxt = i + _NBUF - 1
        if nxt < _NCH:
            nslot = nxt % _NBUF
            if stores[nslot] is not None:
                stores[nslot].wait()
            loads[nslot] = start_load(nxt, nslot)
        loads[slot].wait()
        stores[slot] = start_store(i, slot)
    for j in range(_NBUF):
        stores[j].wait()


_sc_kernel = functools.partial(
    pl.kernel,
    mesh=plsc.VectorSubcoreMesh(core_axis_name="c", subcore_axis_name="s"),
    out_type=jax.ShapeDtypeStruct((_ROWS, _D), jnp.float32),
    scratch_types=(
        [pltpu.VMEM_SHARED((_NS, _NBUF, _CH, _D), jnp.float32)]
        + [pltpu.SemaphoreType.DMA] * (2 * _NBUF)
    ),
)(_sc_copy)


def kernel(x):
    b, s, d = x.shape
    x2 = x.reshape(b * s, d)
    out = _sc_kernel(x2)
    return out.reshape(b, s, d)
